# baseline (speedup 1.0000x reference)
import functools

import jax
import jax.numpy as jnp
from jax import lax
from jax.experimental import pallas as pl
from jax.experimental.pallas import tpu as pltpu
from jax.experimental.pallas import tpu_sc as plsc

N_NODES = 10000
N_EDGES = 320000
D = 128
DH = 64

NC = 2
NS = 16
L = 16

C = 88
NCHUNK = 228
EPT = NCHUNK * C
E_PAD = NS * EPT

WPT = 624
TAIL = N_NODES - NS * WPT
ZR = 104

MROWS = 1000


def _tc_body(x_ref, wt_ref, b_ref, o0_ref, o1_ref):
    y = jnp.dot(x_ref[...], wt_ref[...], preferred_element_type=jnp.float32)
    y = y + b_ref[...]
    o0_ref[...] = y[:, :DH]
    o1_ref[...] = y[:, DH:]


def _tc_linear(layer_input, wt, b2d):
    return pl.pallas_call(
        _tc_body,
        grid=(N_NODES // MROWS,),
        in_specs=[
            pl.BlockSpec((MROWS, D), lambda i: (i, 0)),
            pl.BlockSpec((D, D), lambda i: (0, 0)),
            pl.BlockSpec((1, D), lambda i: (0, 0)),
        ],
        out_specs=[
            pl.BlockSpec((MROWS, DH), lambda i: (i, 0)),
            pl.BlockSpec((MROWS, DH), lambda i: (i, 0)),
        ],
        out_shape=[
            jax.ShapeDtypeStruct((N_NODES, DH), jnp.float32),
            jax.ShapeDtypeStruct((N_NODES, DH), jnp.float32),
        ],
    )(layer_input, wt, b2d)


_sc_mesh = plsc.VectorSubcoreMesh(
    core_axis_name="c", subcore_axis_name="s", num_cores=NC, num_subcores=NS)


@functools.partial(
    pl.kernel,
    out_type=jax.ShapeDtypeStruct((NC, N_NODES, DH), jnp.float32),
    mesh=_sc_mesh,
    compiler_params=pltpu.CompilerParams(
        needs_layout_passes=False, use_tc_tiling_on_sc=False),
    scratch_types=[
        pltpu.VMEM((NCHUNK, C), jnp.int32),
        pltpu.VMEM((NCHUNK, C), jnp.int32),
        pltpu.VMEM((EPT,), jnp.float32),
        pltpu.VMEM((C, DH), jnp.float32),
        pltpu.VMEM((C, DH), jnp.float32),
        pltpu.VMEM((C, DH), jnp.float32),
        pltpu.VMEM((C, DH), jnp.float32),
        pltpu.VMEM((ZR, DH), jnp.float32),
        pltpu.VMEM_SHARED((N_NODES, DH), jnp.float32),
        pltpu.SemaphoreType.DMA,
        pltpu.SemaphoreType.DMA,
        pltpu.SemaphoreType.DMA,
        pltpu.SemaphoreType.DMA,
    ],
)
def _sc_aggregate(x0_hbm, x1_hbm, src_hbm, dst_hbm, adj_hbm, out_hbm,
                  src_v, dst_v, adj_v, rows0_v, rows1_v, scal0_v, scal1_v,
                  zero_v, acc_sh, gsem0, gsem1, ssem0, ssem1):
    c = lax.axis_index("c")
    s = lax.axis_index("s")

    pltpu.sync_copy(src_hbm.at[s], src_v)
    pltpu.sync_copy(dst_hbm.at[s], dst_v)
    pltpu.sync_copy(adj_hbm.at[s], adj_v)

    @plsc.parallel_loop(0, ZR, unroll=8)
    def _zrow(i):
        for k in range(DH // L):
            zero_v[i, pl.ds(k * L, L)] = jnp.zeros((L,), jnp.float32)
    for j in range(WPT // ZR):
        pltpu.sync_copy(zero_v, acc_sh.at[pl.ds(s * WPT + j * ZR, ZR)])

    @pl.when(s == 0)
    def _():
        pltpu.sync_copy(zero_v.at[pl.ds(0, TAIL)],
                        acc_sh.at[pl.ds(NS * WPT, TAIL)])
    plsc.subcore_barrier()

    def _issue(i, buf, gsem):
        @pl.when(c == 0)
        def _():
            pltpu.async_copy(x0_hbm.at[src_v.at[i]], buf, gsem)

        @pl.when(c == 1)
        def _():
            pltpu.async_copy(x1_hbm.at[src_v.at[i]], buf, gsem)

    def _wait_gather(i, buf, gsem):
        @pl.when(c == 0)
        def _():
            pltpu.make_async_copy(x0_hbm.at[src_v.at[i]], buf, gsem).wait()

        @pl.when(c == 1)
        def _():
            pltpu.make_async_copy(x1_hbm.at[src_v.at[i]], buf, gsem).wait()

    def _wait_scatter(i, scal, ssem):
        pltpu.make_async_copy(scal, acc_sh.at[dst_v.at[i]], ssem).wait()

    def _consume(i, buf, scal, gsem, ssem):
        _wait_gather(i, buf, gsem)

        @pl.when(i >= 2)
        def _():
            _wait_scatter(i - 2, scal, ssem)

        @plsc.parallel_loop(0, C, unroll=8)
        def _edge(e):
            a = plsc.load_gather(
                adj_v, [jnp.full((L,), i * C + e, jnp.int32)])
            for k in range(DH // L):
                scal[e, pl.ds(k * L, L)] = buf[e, pl.ds(k * L, L)] * a

        pltpu.async_copy(scal, acc_sh.at[dst_v.at[i]], ssem, add=True)

    _issue(0, rows0_v, gsem0)

    def _chunk(i, carry):
        nxt = i + 1

        @pl.when((nxt < NCHUNK) & (lax.rem(i, 2) == 0))
        def _():
            _issue(nxt, rows1_v, gsem1)

        @pl.when((nxt < NCHUNK) & (lax.rem(i, 2) == 1))
        def _():
            _issue(nxt, rows0_v, gsem0)

        @pl.when(lax.rem(i, 2) == 0)
        def _():
            _consume(i, rows0_v, scal0_v, gsem0, ssem0)

        @pl.when(lax.rem(i, 2) == 1)
        def _():
            _consume(i, rows1_v, scal1_v, gsem1, ssem1)
        return carry
    lax.fori_loop(0, NCHUNK, _chunk, 0)

    _wait_scatter(NCHUNK - 2, (scal0_v, scal1_v)[(NCHUNK - 2) % 2],
                  (ssem0, ssem1)[(NCHUNK - 2) % 2])
    _wait_scatter(NCHUNK - 1, (scal0_v, scal1_v)[(NCHUNK - 1) % 2],
                  (ssem0, ssem1)[(NCHUNK - 1) % 2])

    plsc.subcore_barrier()
    pltpu.sync_copy(acc_sh.at[pl.ds(s * WPT, WPT)],
                    out_hbm.at[c, pl.ds(s * WPT, WPT)])

    @pl.when(s == 0)
    def _():
        pltpu.sync_copy(acc_sh.at[pl.ds(NS * WPT, TAIL)],
                        out_hbm.at[c, pl.ds(NS * WPT, TAIL)])


def kernel(layer_input, edge_index, adj_values, W, b):
    x0, x1 = _tc_linear(layer_input, W.T, b.reshape(1, D))
    ei = edge_index.astype(jnp.int32)
    pad = E_PAD - N_EDGES
    src = jnp.concatenate([ei[1], jnp.zeros((pad,), jnp.int32)])
    dst = jnp.concatenate([ei[0], jnp.zeros((pad,), jnp.int32)])
    adjp = jnp.concatenate([adj_values, jnp.zeros((pad,), jnp.float32)])
    src = src.reshape(NS, NCHUNK, C)
    dst = dst.reshape(NS, NCHUNK, C)
    adj = adjp.reshape(NS, EPT)
    halves = _sc_aggregate(x0, x1, src, dst, adj)
    return jnp.concatenate([halves[0], halves[1]], axis=1)


# speedup vs baseline: 1.1585x; 1.1585x over previous
import functools

import numpy as np

import jax
import jax.numpy as jnp
from jax import lax
from jax.experimental import pallas as pl
from jax.experimental.pallas import tpu as pltpu
from jax.experimental.pallas import tpu_sc as plsc

N_NODES = 10000
N_EDGES = 320000
D = 128
DH = 64

NC = 2
NS = 16
L = 16

C = 80
NCHUNK = 250
EPT = NCHUNK * C
E_PAD = NS * EPT

WPT = 624
TAIL = N_NODES - NS * WPT
ZR = 104

MROWS = 1000

_ph = np.empty(64, np.int32)
for _k in range(2):
    for _j in range(16):
        _ph[32 * _k + 2 * _j] = 32 * _k + _j
        _ph[32 * _k + 2 * _j + 1] = 32 * _k + 16 + _j
PERM = np.concatenate([_ph, 64 + _ph])


def _tc_body(x_ref, wt_ref, b_ref, o0_ref, o1_ref):
    y = jnp.dot(x_ref[...], wt_ref[...], preferred_element_type=jnp.float32)
    y = y + b_ref[...]
    o0_ref[...] = y[:, :DH].astype(jnp.bfloat16)
    o1_ref[...] = y[:, DH:].astype(jnp.bfloat16)


def _tc_linear(layer_input, wt, b2d):
    return pl.pallas_call(
        _tc_body,
        grid=(N_NODES // MROWS,),
        in_specs=[
            pl.BlockSpec((MROWS, D), lambda i: (i, 0)),
            pl.BlockSpec((D, D), lambda i: (0, 0)),
            pl.BlockSpec((1, D), lambda i: (0, 0)),
        ],
        out_specs=[
            pl.BlockSpec((MROWS, DH), lambda i: (i, 0)),
            pl.BlockSpec((MROWS, DH), lambda i: (i, 0)),
        ],
        out_shape=[
            jax.ShapeDtypeStruct((N_NODES, DH), jnp.bfloat16),
            jax.ShapeDtypeStruct((N_NODES, DH), jnp.bfloat16),
        ],
    )(layer_input, wt, b2d)


_sc_mesh = plsc.VectorSubcoreMesh(
    core_axis_name="c", subcore_axis_name="s", num_cores=NC, num_subcores=NS)


@functools.partial(
    pl.kernel,
    out_type=jax.ShapeDtypeStruct((NC, N_NODES, DH), jnp.float32),
    mesh=_sc_mesh,
    compiler_params=pltpu.CompilerParams(
        needs_layout_passes=False, use_tc_tiling_on_sc=False),
    scratch_types=[
        pltpu.VMEM((NCHUNK, C), jnp.int32),
        pltpu.VMEM((NCHUNK, C), jnp.int32),
        pltpu.VMEM((EPT,), jnp.float32),
        pltpu.VMEM((C, DH), jnp.bfloat16),
        pltpu.VMEM((C, DH), jnp.bfloat16),
        pltpu.VMEM((C, DH), jnp.float32),
        pltpu.VMEM((C, DH), jnp.float32),
        pltpu.VMEM((ZR, DH), jnp.float32),
        pltpu.VMEM_SHARED((N_NODES, DH), jnp.float32),
        pltpu.SemaphoreType.DMA,
        pltpu.SemaphoreType.DMA,
        pltpu.SemaphoreType.DMA,
        pltpu.SemaphoreType.DMA,
    ],
)
def _sc_aggregate(x0_hbm, x1_hbm, src_hbm, dst_hbm, adj_hbm, out_hbm,
                  src_v, dst_v, adj_v, rows0_v, rows1_v, scal0_v, scal1_v,
                  zero_v, acc_sh, gsem0, gsem1, ssem0, ssem1):
    c = lax.axis_index("c")
    s = lax.axis_index("s")

    pltpu.sync_copy(src_hbm.at[s], src_v)
    pltpu.sync_copy(dst_hbm.at[s], dst_v)
    pltpu.sync_copy(adj_hbm.at[s], adj_v)

    @plsc.parallel_loop(0, ZR, unroll=8)
    def _zrow(i):
        for k in range(DH // L):
            zero_v[i, pl.ds(k * L, L)] = jnp.zeros((L,), jnp.float32)
    for j in range(WPT // ZR):
        pltpu.sync_copy(zero_v, acc_sh.at[pl.ds(s * WPT + j * ZR, ZR)])

    @pl.when(s == 0)
    def _():
        pltpu.sync_copy(zero_v.at[pl.ds(0, TAIL)],
                        acc_sh.at[pl.ds(NS * WPT, TAIL)])
    plsc.subcore_barrier()

    def _issue(i, buf, gsem):
        @pl.when(c == 0)
        def _():
            pltpu.async_copy(x0_hbm.at[src_v.at[i]], buf, gsem)

        @pl.when(c == 1)
        def _():
            pltpu.async_copy(x1_hbm.at[src_v.at[i]], buf, gsem)

    def _wait_gather(i, buf, gsem):
        @pl.when(c == 0)
        def _():
            pltpu.make_async_copy(x0_hbm.at[src_v.at[i]], buf, gsem).wait()

        @pl.when(c == 1)
        def _():
            pltpu.make_async_copy(x1_hbm.at[src_v.at[i]], buf, gsem).wait()

    def _wait_scatter(i, scal, ssem):
        pltpu.make_async_copy(scal, acc_sh.at[dst_v.at[i]], ssem).wait()

    def _consume(i, buf, scal, gsem, ssem):
        _wait_gather(i, buf, gsem)

        @pl.when(i >= 2)
        def _():
            _wait_scatter(i - 2, scal, ssem)

        @plsc.parallel_loop(0, C, unroll=8)
        def _edge(e):
            a = plsc.load_gather(
                adj_v, [jnp.full((L,), i * C + e, jnp.int32)])
            for k in range(DH // (2 * L)):
                v = buf[e, pl.ds(k * 2 * L, 2 * L)]
                lo, hi = plsc.unpack(
                    v, format=plsc.PackFormat.INTERLEAVED,
                    preferred_element_type=jnp.float32)
                scal[e, pl.ds(k * 2 * L, L)] = lo * a
                scal[e, pl.ds(k * 2 * L + L, L)] = hi * a

        pltpu.async_copy(scal, acc_sh.at[dst_v.at[i]], ssem, add=True)

    _issue(0, rows0_v, gsem0)

    def _chunk(i, carry):
        nxt = i + 1

        @pl.when((nxt < NCHUNK) & (lax.rem(i, 2) == 0))
        def _():
            _issue(nxt, rows1_v, gsem1)

        @pl.when((nxt < NCHUNK) & (lax.rem(i, 2) == 1))
        def _():
            _issue(nxt, rows0_v, gsem0)

        @pl.when(lax.rem(i, 2) == 0)
        def _():
            _consume(i, rows0_v, scal0_v, gsem0, ssem0)

        @pl.when(lax.rem(i, 2) == 1)
        def _():
            _consume(i, rows1_v, scal1_v, gsem1, ssem1)
        return carry
    lax.fori_loop(0, NCHUNK, _chunk, 0)

    _wait_scatter(NCHUNK - 2, (scal0_v, scal1_v)[(NCHUNK - 2) % 2],
                  (ssem0, ssem1)[(NCHUNK - 2) % 2])
    _wait_scatter(NCHUNK - 1, (scal0_v, scal1_v)[(NCHUNK - 1) % 2],
                  (ssem0, ssem1)[(NCHUNK - 1) % 2])

    plsc.subcore_barrier()
    pltpu.sync_copy(acc_sh.at[pl.ds(s * WPT, WPT)],
                    out_hbm.at[c, pl.ds(s * WPT, WPT)])

    @pl.when(s == 0)
    def _():
        pltpu.sync_copy(acc_sh.at[pl.ds(NS * WPT, TAIL)],
                        out_hbm.at[c, pl.ds(NS * WPT, TAIL)])


def kernel(layer_input, edge_index, adj_values, W, b):
    x0, x1 = _tc_linear(layer_input, W[PERM].T, b[PERM].reshape(1, D))
    ei = edge_index.astype(jnp.int32)
    src = ei[1].reshape(NS, NCHUNK, C)
    dst = ei[0].reshape(NS, NCHUNK, C)
    adj = adj_values.reshape(NS, EPT)
    halves = _sc_aggregate(x0, x1, src, dst, adj)
    return jnp.concatenate([halves[0], halves[1]], axis=1)


# speedup vs baseline: 1.1943x; 1.0309x over previous
import functools

import numpy as np

import jax
import jax.numpy as jnp
from jax import lax
from jax.experimental import pallas as pl
from jax.experimental.pallas import tpu as pltpu
from jax.experimental.pallas import tpu_sc as plsc

N_NODES = 10000
N_EDGES = 320000
D = 128
DH = 64

NC = 2
NS = 16
L = 16

C = 80
NBUF = 4
NCHUNK = 252
EPT = NCHUNK * C
E_PAD = NS * EPT
NGRP = NCHUNK // NBUF

WPT = 624
TAIL = N_NODES - NS * WPT
ZR = 104

MROWS = 1000

_ph = np.empty(64, np.int32)
for _k in range(2):
    for _j in range(16):
        _ph[32 * _k + 2 * _j] = 32 * _k + _j
        _ph[32 * _k + 2 * _j + 1] = 32 * _k + 16 + _j
PERM = np.concatenate([_ph, 64 + _ph])


def _tc_body(x_ref, wt_ref, b_ref, o0_ref, o1_ref):
    y = jnp.dot(x_ref[...], wt_ref[...], preferred_element_type=jnp.float32)
    y = y + b_ref[...]
    o0_ref[...] = y[:, :DH].astype(jnp.bfloat16)
    o1_ref[...] = y[:, DH:].astype(jnp.bfloat16)


def _tc_linear(layer_input, wt, b2d):
    return pl.pallas_call(
        _tc_body,
        grid=(N_NODES // MROWS,),
        in_specs=[
            pl.BlockSpec((MROWS, D), lambda i: (i, 0)),
            pl.BlockSpec((D, D), lambda i: (0, 0)),
            pl.BlockSpec((1, D), lambda i: (0, 0)),
        ],
        out_specs=[
            pl.BlockSpec((MROWS, DH), lambda i: (i, 0)),
            pl.BlockSpec((MROWS, DH), lambda i: (i, 0)),
        ],
        out_shape=[
            jax.ShapeDtypeStruct((N_NODES, DH), jnp.bfloat16),
            jax.ShapeDtypeStruct((N_NODES, DH), jnp.bfloat16),
        ],
    )(layer_input, wt, b2d)


_sc_mesh = plsc.VectorSubcoreMesh(
    core_axis_name="c", subcore_axis_name="s", num_cores=NC, num_subcores=NS)


@functools.partial(
    pl.kernel,
    out_type=jax.ShapeDtypeStruct((NC, N_NODES, DH), jnp.float32),
    mesh=_sc_mesh,
    compiler_params=pltpu.CompilerParams(
        needs_layout_passes=False, use_tc_tiling_on_sc=False),
    scratch_types=[
        pltpu.VMEM((NCHUNK, C), jnp.int32),
        pltpu.VMEM((NCHUNK, C), jnp.int32),
        pltpu.VMEM((EPT,), jnp.float32),
        [pltpu.VMEM((C, DH), jnp.bfloat16) for _ in range(NBUF)],
        [pltpu.VMEM((C, DH), jnp.float32) for _ in range(2)],
        pltpu.VMEM((ZR, DH), jnp.float32),
        pltpu.VMEM_SHARED((N_NODES, DH), jnp.float32),
        [pltpu.SemaphoreType.DMA for _ in range(NBUF)],
        [pltpu.SemaphoreType.DMA for _ in range(2)],
    ],
)
def _sc_aggregate(x0_hbm, x1_hbm, src_hbm, dst_hbm, adj_hbm, out_hbm,
                  src_v, dst_v, adj_v, rows_v, scal_v, zero_v, acc_sh,
                  gsem, ssem):
    c = lax.axis_index("c")
    s = lax.axis_index("s")

    pltpu.sync_copy(src_hbm.at[s], src_v)
    pltpu.sync_copy(dst_hbm.at[s], dst_v)
    pltpu.sync_copy(adj_hbm.at[s], adj_v)

    @plsc.parallel_loop(0, ZR, unroll=8)
    def _zrow(i):
        for k in range(DH // L):
            zero_v[i, pl.ds(k * L, L)] = jnp.zeros((L,), jnp.float32)
    for j in range(WPT // ZR):
        pltpu.sync_copy(zero_v, acc_sh.at[pl.ds(s * WPT + j * ZR, ZR)])

    @pl.when(s == 0)
    def _():
        pltpu.sync_copy(zero_v.at[pl.ds(0, TAIL)],
                        acc_sh.at[pl.ds(NS * WPT, TAIL)])
    plsc.subcore_barrier()

    def _issue(i, buf, sem):
        @pl.when(c == 0)
        def _():
            pltpu.async_copy(x0_hbm.at[src_v.at[i]], buf, sem)

        @pl.when(c == 1)
        def _():
            pltpu.async_copy(x1_hbm.at[src_v.at[i]], buf, sem)

    def _wait_gather(i, buf, sem):
        @pl.when(c == 0)
        def _():
            pltpu.make_async_copy(x0_hbm.at[src_v.at[i]], buf, sem).wait()

        @pl.when(c == 1)
        def _():
            pltpu.make_async_copy(x1_hbm.at[src_v.at[i]], buf, sem).wait()

    def _wait_scatter(i, scal, sem):
        pltpu.make_async_copy(scal, acc_sh.at[dst_v.at[i]], sem).wait()

    for b in range(NBUF):
        _issue(b, rows_v[b], gsem[b])

    def _group(g, carry):
        for b in range(NBUF):
            i = g * NBUF + b
            _wait_gather(i, rows_v[b], gsem[b])

            if b >= 2:
                _wait_scatter(i - 2, scal_v[b % 2], ssem[b % 2])
            else:
                @pl.when(g > 0)
                def _(i=i, b=b):
                    _wait_scatter(i - 2, scal_v[b % 2], ssem[b % 2])

            buf = rows_v[b]
            scal = scal_v[b % 2]

            @plsc.parallel_loop(0, C, unroll=8)
            def _edge(e, i=i, buf=buf, scal=scal):
                a = plsc.load_gather(
                    adj_v, [jnp.full((L,), i * C + e, jnp.int32)])
                for k in range(DH // (2 * L)):
                    v = buf[e, pl.ds(k * 2 * L, 2 * L)]
                    lo, hi = plsc.unpack(
                        v, format=plsc.PackFormat.INTERLEAVED,
                        preferred_element_type=jnp.float32)
                    scal[e, pl.ds(k * 2 * L, L)] = lo * a
                    scal[e, pl.ds(k * 2 * L + L, L)] = hi * a

            pltpu.async_copy(scal, acc_sh.at[dst_v.at[i]], ssem[b % 2],
                             add=True)

            @pl.when(g < NGRP - 1)
            def _(i=i, b=b):
                _issue(i + NBUF, rows_v[b], gsem[b])
        return carry
    lax.fori_loop(0, NGRP, _group, 0)

    for b in range(NBUF - 2, NBUF):
        _wait_scatter((NGRP - 1) * NBUF + b, scal_v[b % 2], ssem[b % 2])

    plsc.subcore_barrier()
    pltpu.sync_copy(acc_sh.at[pl.ds(s * WPT, WPT)],
                    out_hbm.at[c, pl.ds(s * WPT, WPT)])

    @pl.when(s == 0)
    def _():
        pltpu.sync_copy(acc_sh.at[pl.ds(NS * WPT, TAIL)],
                        out_hbm.at[c, pl.ds(NS * WPT, TAIL)])


def kernel(layer_input, edge_index, adj_values, W, b):
    x0, x1 = _tc_linear(layer_input, W[PERM].T, b[PERM].reshape(1, D))
    ei = edge_index.astype(jnp.int32)
    pad = E_PAD - N_EDGES
    src = jnp.concatenate([ei[1], jnp.zeros((pad,), jnp.int32)])
    dst = jnp.concatenate([ei[0], jnp.zeros((pad,), jnp.int32)])
    adjp = jnp.concatenate([adj_values, jnp.zeros((pad,), jnp.float32)])
    halves = _sc_aggregate(x0, x1,
                           src.reshape(NS, NCHUNK, C),
                           dst.reshape(NS, NCHUNK, C),
                           adjp.reshape(NS, EPT))
    return jnp.concatenate([halves[0], halves[1]], axis=1)


# speedup vs baseline: 1.2937x; 1.0833x over previous
import functools

import numpy as np

import jax
import jax.numpy as jnp
from jax import lax
from jax.experimental import pallas as pl
from jax.experimental.pallas import tpu as pltpu
from jax.experimental.pallas import tpu_sc as plsc

N_NODES = 10000
N_EDGES = 320000
D = 128
DH = 64

NC = 2
NS = 16
L = 16

C = 80
NBUF = 4
NCHUNK = 252
EPT = NCHUNK * C
E_PAD = NS * EPT
NGRP = NCHUNK // NBUF

WPT = 624
TAIL = N_NODES - NS * WPT
ZR = 104

MROWS = 1000

_ph = np.empty(64, np.int32)
for _k in range(2):
    for _j in range(16):
        _ph[32 * _k + 2 * _j] = 32 * _k + _j
        _ph[32 * _k + 2 * _j + 1] = 32 * _k + 16 + _j
PERM = np.concatenate([_ph, 64 + _ph])


def _tc_body(x_ref, wt_ref, b_ref, o0_ref, o1_ref):
    y = jnp.dot(x_ref[...], wt_ref[...], preferred_element_type=jnp.float32)
    y = y + b_ref[...]
    o0_ref[...] = y[:, :DH].astype(jnp.bfloat16)
    o1_ref[...] = y[:, DH:].astype(jnp.bfloat16)


def _tc_linear(layer_input, wt, b2d):
    return pl.pallas_call(
        _tc_body,
        grid=(N_NODES // MROWS,),
        in_specs=[
            pl.BlockSpec((MROWS, D), lambda i: (i, 0)),
            pl.BlockSpec((D, D), lambda i: (0, 0)),
            pl.BlockSpec((1, D), lambda i: (0, 0)),
        ],
        out_specs=[
            pl.BlockSpec((MROWS, DH), lambda i: (i, 0)),
            pl.BlockSpec((MROWS, DH), lambda i: (i, 0)),
        ],
        out_shape=[
            jax.ShapeDtypeStruct((N_NODES, DH), jnp.bfloat16),
            jax.ShapeDtypeStruct((N_NODES, DH), jnp.bfloat16),
        ],
    )(layer_input, wt, b2d)


_sc_mesh = plsc.VectorSubcoreMesh(
    core_axis_name="c", subcore_axis_name="s", num_cores=NC, num_subcores=NS)


@functools.partial(
    pl.kernel,
    out_type=jax.ShapeDtypeStruct((N_NODES, D), jnp.float32),
    mesh=_sc_mesh,
    compiler_params=pltpu.CompilerParams(
        needs_layout_passes=False, use_tc_tiling_on_sc=False),
    scratch_types=[
        pltpu.VMEM((NCHUNK, C), jnp.int32),
        pltpu.VMEM((NCHUNK, C), jnp.int32),
        pltpu.VMEM((EPT,), jnp.float32),
        [pltpu.VMEM((C, DH), jnp.bfloat16) for _ in range(NBUF)],
        [pltpu.VMEM((C, DH), jnp.float32) for _ in range(2)],
        pltpu.VMEM((ZR, DH), jnp.float32),
        pltpu.VMEM_SHARED((N_NODES, DH), jnp.float32),
        [pltpu.SemaphoreType.DMA for _ in range(NBUF)],
        [pltpu.SemaphoreType.DMA for _ in range(2)],
    ],
)
def _sc_aggregate(x0_hbm, x1_hbm, src_hbm, dst_hbm, adj_hbm, out_hbm,
                  src_v, dst_v, adj_v, rows_v, scal_v, zero_v, acc_sh,
                  gsem, ssem):
    c = lax.axis_index("c")
    s = lax.axis_index("s")

    pltpu.sync_copy(src_hbm.at[s], src_v)
    pltpu.sync_copy(dst_hbm.at[s], dst_v)
    pltpu.sync_copy(adj_hbm.at[s], adj_v)

    @plsc.parallel_loop(0, ZR, unroll=8)
    def _zrow(i):
        for k in range(DH // L):
            zero_v[i, pl.ds(k * L, L)] = jnp.zeros((L,), jnp.float32)
    for j in range(WPT // ZR):
        pltpu.sync_copy(zero_v, acc_sh.at[pl.ds(s * WPT + j * ZR, ZR)])

    @pl.when(s == 0)
    def _():
        pltpu.sync_copy(zero_v.at[pl.ds(0, TAIL)],
                        acc_sh.at[pl.ds(NS * WPT, TAIL)])
    plsc.subcore_barrier()

    def _issue(i, buf, sem):
        @pl.when(c == 0)
        def _():
            pltpu.async_copy(x0_hbm.at[src_v.at[i]], buf, sem)

        @pl.when(c == 1)
        def _():
            pltpu.async_copy(x1_hbm.at[src_v.at[i]], buf, sem)

    def _wait_gather(i, buf, sem):
        @pl.when(c == 0)
        def _():
            pltpu.make_async_copy(x0_hbm.at[src_v.at[i]], buf, sem).wait()

        @pl.when(c == 1)
        def _():
            pltpu.make_async_copy(x1_hbm.at[src_v.at[i]], buf, sem).wait()

    def _wait_scatter(i, scal, sem):
        pltpu.make_async_copy(scal, acc_sh.at[dst_v.at[i]], sem).wait()

    for b in range(NBUF):
        _issue(b, rows_v[b], gsem[b])

    def _group(g, carry):
        for b in range(NBUF):
            i = g * NBUF + b
            _wait_gather(i, rows_v[b], gsem[b])

            if b >= 2:
                _wait_scatter(i - 2, scal_v[b % 2], ssem[b % 2])
            else:
                @pl.when(g > 0)
                def _(i=i, b=b):
                    _wait_scatter(i - 2, scal_v[b % 2], ssem[b % 2])

            buf = rows_v[b]
            scal = scal_v[b % 2]

            @plsc.parallel_loop(0, C, unroll=8)
            def _edge(e, i=i, buf=buf, scal=scal):
                a = plsc.load_gather(
                    adj_v, [jnp.full((L,), i * C + e, jnp.int32)])
                for k in range(DH // (2 * L)):
                    v = buf[e, pl.ds(k * 2 * L, 2 * L)]
                    lo, hi = plsc.unpack(
                        v, format=plsc.PackFormat.INTERLEAVED,
                        preferred_element_type=jnp.float32)
                    scal[e, pl.ds(k * 2 * L, L)] = lo * a
                    scal[e, pl.ds(k * 2 * L + L, L)] = hi * a

            pltpu.async_copy(scal, acc_sh.at[dst_v.at[i]], ssem[b % 2],
                             add=True)

            @pl.when(g < NGRP - 1)
            def _(i=i, b=b):
                _issue(i + NBUF, rows_v[b], gsem[b])
        return carry
    lax.fori_loop(0, NGRP, _group, 0)

    for b in range(NBUF - 2, NBUF):
        _wait_scatter((NGRP - 1) * NBUF + b, scal_v[b % 2], ssem[b % 2])

    plsc.subcore_barrier()
    pltpu.sync_copy(acc_sh.at[pl.ds(s * WPT, WPT)],
                    out_hbm.at[pl.ds(s * WPT, WPT), pl.ds(c * DH, DH)])

    @pl.when(s == 0)
    def _():
        pltpu.sync_copy(acc_sh.at[pl.ds(NS * WPT, TAIL)],
                        out_hbm.at[pl.ds(NS * WPT, TAIL), pl.ds(c * DH, DH)])


def kernel(layer_input, edge_index, adj_values, W, b):
    x0, x1 = _tc_linear(layer_input, W[PERM].T, b[PERM].reshape(1, D))
    ei = edge_index.astype(jnp.int32)
    pad = E_PAD - N_EDGES
    src = jnp.concatenate([ei[1], jnp.zeros((pad,), jnp.int32)])
    dst = jnp.concatenate([ei[0], jnp.zeros((pad,), jnp.int32)])
    adjp = jnp.concatenate([adj_values, jnp.zeros((pad,), jnp.float32)])
    return _sc_aggregate(x0, x1,
                         src.reshape(NS, NCHUNK, C),
                         dst.reshape(NS, NCHUNK, C),
                         adjp.reshape(NS, EPT))


# speedup vs baseline: 1.5043x; 1.1627x over previous
import functools

import numpy as np

import jax
import jax.numpy as jnp
from jax import lax
from jax.experimental import pallas as pl
from jax.experimental.pallas import tpu as pltpu
from jax.experimental.pallas import tpu_sc as plsc

N_NODES = 10000
N_EDGES = 320000
D = 128
DH = 64

NC = 2
NS = 16
L = 16

C = 80
NBUF = 4
NCHUNK = 250
EPT = NCHUNK * C
NGRP = NCHUNK // NBUF

WPT = 624
TAIL = N_NODES - NS * WPT
ZR = 104

MROWS = 1000

_ph = np.empty(64, np.int32)
for _k in range(2):
    for _j in range(16):
        _ph[32 * _k + 2 * _j] = 32 * _k + _j
        _ph[32 * _k + 2 * _j + 1] = 32 * _k + 16 + _j
PERM = np.concatenate([_ph, 64 + _ph])


def _tc_body(x_ref, wt_ref, b_ref, o0_ref, o1_ref):
    y = jnp.dot(x_ref[...], wt_ref[...], preferred_element_type=jnp.float32)
    y = y + b_ref[...]
    o0_ref[...] = y[:, :DH].astype(jnp.bfloat16)
    o1_ref[...] = y[:, DH:].astype(jnp.bfloat16)


def _tc_linear(layer_input, wt, b2d):
    return pl.pallas_call(
        _tc_body,
        grid=(N_NODES // MROWS,),
        in_specs=[
            pl.BlockSpec((MROWS, D), lambda i: (i, 0)),
            pl.BlockSpec((D, D), lambda i: (0, 0)),
            pl.BlockSpec((1, D), lambda i: (0, 0)),
        ],
        out_specs=[
            pl.BlockSpec((MROWS, DH), lambda i: (i, 0)),
            pl.BlockSpec((MROWS, DH), lambda i: (i, 0)),
        ],
        out_shape=[
            jax.ShapeDtypeStruct((N_NODES, DH), jnp.bfloat16),
            jax.ShapeDtypeStruct((N_NODES, DH), jnp.bfloat16),
        ],
    )(layer_input, wt, b2d)


_sc_mesh = plsc.VectorSubcoreMesh(
    core_axis_name="c", subcore_axis_name="s", num_cores=NC, num_subcores=NS)


@functools.partial(
    pl.kernel,
    out_type=jax.ShapeDtypeStruct((N_NODES, D), jnp.float32),
    mesh=_sc_mesh,
    compiler_params=pltpu.CompilerParams(
        needs_layout_passes=False, use_tc_tiling_on_sc=False),
    scratch_types=[
        pltpu.VMEM((NCHUNK, C), jnp.int32),
        pltpu.VMEM((NCHUNK, C), jnp.int32),
        pltpu.VMEM((EPT,), jnp.float32),
        [pltpu.VMEM((C, DH), jnp.bfloat16) for _ in range(NBUF)],
        [pltpu.VMEM((C, DH), jnp.float32) for _ in range(2)],
        pltpu.VMEM((ZR, DH), jnp.float32),
        pltpu.VMEM_SHARED((N_NODES, DH), jnp.float32),
        [pltpu.SemaphoreType.DMA for _ in range(NBUF)],
        [pltpu.SemaphoreType.DMA for _ in range(2)],
    ],
)
def _sc_aggregate(x0_hbm, x1_hbm, src_hbm, dst_hbm, adj_hbm, out_hbm,
                  src_v, dst_v, adj_v, rows_v, scal_v, zero_v, acc_sh,
                  gsem, ssem):
    c = lax.axis_index("c")
    s = lax.axis_index("s")

    pltpu.sync_copy(src_hbm.at[s], src_v)
    pltpu.sync_copy(dst_hbm.at[s], dst_v)
    pltpu.sync_copy(adj_hbm.at[s], adj_v)

    @plsc.parallel_loop(0, ZR, unroll=8)
    def _zrow(i):
        for k in range(DH // L):
            zero_v[i, pl.ds(k * L, L)] = jnp.zeros((L,), jnp.float32)
    for j in range(WPT // ZR):
        pltpu.sync_copy(zero_v, acc_sh.at[pl.ds(s * WPT + j * ZR, ZR)])

    @pl.when(s == 0)
    def _():
        pltpu.sync_copy(zero_v.at[pl.ds(0, TAIL)],
                        acc_sh.at[pl.ds(NS * WPT, TAIL)])
    plsc.subcore_barrier()

    def _issue(i, buf, sem):
        @pl.when(c == 0)
        def _():
            pltpu.async_copy(x0_hbm.at[src_v.at[i]], buf, sem)

        @pl.when(c == 1)
        def _():
            pltpu.async_copy(x1_hbm.at[src_v.at[i]], buf, sem)

    def _wait_gather(i, buf, sem):
        @pl.when(c == 0)
        def _():
            pltpu.make_async_copy(x0_hbm.at[src_v.at[i]], buf, sem).wait()

        @pl.when(c == 1)
        def _():
            pltpu.make_async_copy(x1_hbm.at[src_v.at[i]], buf, sem).wait()

    def _wait_scatter(i, scal, sem):
        pltpu.make_async_copy(scal, acc_sh.at[dst_v.at[i]], sem).wait()

    for b in range(NBUF):
        _issue(b, rows_v[b], gsem[b])

    def _group(g, carry):
        for b in range(NBUF):
            i = g * NBUF + b
            _wait_gather(i, rows_v[b], gsem[b])

            if b >= 2:
                _wait_scatter(i - 2, scal_v[b % 2], ssem[b % 2])
            else:
                @pl.when(g > 0)
                def _(i=i, b=b):
                    _wait_scatter(i - 2, scal_v[b % 2], ssem[b % 2])

            buf = rows_v[b]
            scal = scal_v[b % 2]

            @plsc.parallel_loop(0, C, unroll=8)
            def _edge(e, i=i, buf=buf, scal=scal):
                a = plsc.load_gather(
                    adj_v, [jnp.full((L,), i * C + e, jnp.int32)])
                for k in range(DH // (2 * L)):
                    v = buf[e, pl.ds(k * 2 * L, 2 * L)]
                    lo, hi = plsc.unpack(
                        v, format=plsc.PackFormat.INTERLEAVED,
                        preferred_element_type=jnp.float32)
                    scal[e, pl.ds(k * 2 * L, L)] = lo * a
                    scal[e, pl.ds(k * 2 * L + L, L)] = hi * a

            pltpu.async_copy(scal, acc_sh.at[dst_v.at[i]], ssem[b % 2],
                             add=True)

            if b < NCHUNK - NGRP * NBUF:
                _issue(i + NBUF, rows_v[b], gsem[b])
            else:
                @pl.when(g < NGRP - 1)
                def _(i=i, b=b):
                    _issue(i + NBUF, rows_v[b], gsem[b])
        return carry
    lax.fori_loop(0, NGRP, _group, 0)

    for b in range(NCHUNK - NGRP * NBUF):
        i = NGRP * NBUF + b
        _wait_gather(i, rows_v[b], gsem[b])
        _wait_scatter(i - 2, scal_v[b % 2], ssem[b % 2])
        buf = rows_v[b]
        scal = scal_v[b % 2]

        @plsc.parallel_loop(0, C, unroll=8)
        def _edge(e, i=i, buf=buf, scal=scal):
            a = plsc.load_gather(
                adj_v, [jnp.full((L,), i * C + e, jnp.int32)])
            for k in range(DH // (2 * L)):
                v = buf[e, pl.ds(k * 2 * L, 2 * L)]
                lo, hi = plsc.unpack(
                    v, format=plsc.PackFormat.INTERLEAVED,
                    preferred_element_type=jnp.float32)
                scal[e, pl.ds(k * 2 * L, L)] = lo * a
                scal[e, pl.ds(k * 2 * L + L, L)] = hi * a

        pltpu.async_copy(scal, acc_sh.at[dst_v.at[i]], ssem[b % 2], add=True)

    for i in (NCHUNK - 2, NCHUNK - 1):
        _wait_scatter(i, scal_v[i % 2], ssem[i % 2])

    plsc.subcore_barrier()
    pltpu.sync_copy(acc_sh.at[pl.ds(s * WPT, WPT)],
                    out_hbm.at[pl.ds(s * WPT, WPT), pl.ds(c * DH, DH)])

    @pl.when(s == 0)
    def _():
        pltpu.sync_copy(acc_sh.at[pl.ds(NS * WPT, TAIL)],
                        out_hbm.at[pl.ds(NS * WPT, TAIL), pl.ds(c * DH, DH)])


def kernel(layer_input, edge_index, adj_values, W, b):
    x0, x1 = _tc_linear(layer_input, W[PERM].T, b[PERM].reshape(1, D))
    ei = edge_index.astype(jnp.int32)
    return _sc_aggregate(x0, x1,
                         ei[1].reshape(NS, NCHUNK, C),
                         ei[0].reshape(NS, NCHUNK, C),
                         adj_values.reshape(NS, EPT))
